# baseline (device time: 4249960 ns/iter reference)
import jax
import jax.numpy as jnp
from jax import lax
from jax.experimental import pallas as pl
from jax.experimental.pallas import tpu as pltpu

N_CHUNKS = 16
N_LOCAL = 8


def kernel(x):
    m_per, n = x.shape
    m_global = 2 * m_per
    half = m_per // 2
    c = half // N_CHUNKS
    lc = m_per // N_LOCAL

    def body(x_ref, out_ref, copy_sems, x_send, x_recv, y_send, y_recv):
        my_x = lax.axis_index("x")
        my_y = lax.axis_index("y")
        xn = (1 - my_x, my_y)
        yn = (my_x, 1 - my_y)

        barrier_sem = pltpu.get_barrier_semaphore()
        for nbr in (xn, yn):
            pl.semaphore_signal(
                barrier_sem, inc=1, device_id=nbr,
                device_id_type=pl.DeviceIdType.MESH,
            )
        pl.semaphore_wait(barrier_sem, 2)

        local_copies = []
        for k in range(N_LOCAL):
            cp = pltpu.make_async_copy(
                x_ref.at[pl.ds(k * lc, lc)],
                out_ref.at[pl.ds(my_x * m_per + k * lc, lc)],
                copy_sems.at[k],
            )
            cp.start()
            local_copies.append(cp)

        x_rdmas = []
        for k in range(N_CHUNKS):
            src_row = my_y * half + k * c
            dst_row = my_x * m_per + my_y * half + k * c
            r = pltpu.make_async_remote_copy(
                src_ref=x_ref.at[pl.ds(src_row, c)],
                dst_ref=out_ref.at[pl.ds(dst_row, c)],
                send_sem=x_send.at[k],
                recv_sem=x_recv.at[k],
                device_id=xn,
                device_id_type=pl.DeviceIdType.MESH,
            )
            r.start()
            x_rdmas.append(r)

        y_rdmas = []
        for k in range(N_CHUNKS):
            x_rdmas[k].wait_recv()
            row = (1 - my_x) * m_per + my_y * half + k * c
            r = pltpu.make_async_remote_copy(
                src_ref=out_ref.at[pl.ds(row, c)],
                dst_ref=out_ref.at[pl.ds(row, c)],
                send_sem=y_send.at[k],
                recv_sem=y_recv.at[k],
                device_id=yn,
                device_id_type=pl.DeviceIdType.MESH,
            )
            r.start()
            y_rdmas.append(r)

        for k in range(N_CHUNKS):
            x_rdmas[k].wait_send()
            y_rdmas[k].wait_send()
            y_rdmas[k].wait_recv()
        for cp in local_copies:
            cp.wait()

    return pl.pallas_call(
        body,
        out_shape=jax.ShapeDtypeStruct((m_global, n), x.dtype),
        in_specs=[pl.BlockSpec(memory_space=pl.ANY)],
        out_specs=pl.BlockSpec(memory_space=pl.ANY),
        scratch_shapes=[
            pltpu.SemaphoreType.DMA((N_LOCAL,)),
            pltpu.SemaphoreType.DMA((N_CHUNKS,)),
            pltpu.SemaphoreType.DMA((N_CHUNKS,)),
            pltpu.SemaphoreType.DMA((N_CHUNKS,)),
            pltpu.SemaphoreType.DMA((N_CHUNKS,)),
        ],
        compiler_params=pltpu.CompilerParams(collective_id=0),
    )(x)


# device time: 942404 ns/iter; 4.5097x vs baseline; 4.5097x over previous
import jax
import jax.numpy as jnp
from jax import lax
from jax.experimental import pallas as pl
from jax.experimental.pallas import tpu as pltpu

N_CHUNKS = 16


def kernel(x):
    m_per, n = x.shape
    m_global = 2 * m_per
    half = m_per // 2
    c = half // N_CHUNKS
    lc = m_per // N_CHUNKS

    def body(x_ref, out_ref, stage, in_sems, out_sems,
             x_send, x_recv, y_send, y_recv):
        my_x = lax.axis_index("x")
        my_y = lax.axis_index("y")
        xn = (1 - my_x, my_y)
        yn = (my_x, 1 - my_y)

        barrier_sem = pltpu.get_barrier_semaphore()
        for nbr in (xn, yn):
            pl.semaphore_signal(
                barrier_sem, inc=1, device_id=nbr,
                device_id_type=pl.DeviceIdType.MESH,
            )
        pl.semaphore_wait(barrier_sem, 2)

        x_rdmas = []
        for k in range(N_CHUNKS):
            src_row = my_y * half + k * c
            dst_row = my_x * m_per + my_y * half + k * c
            r = pltpu.make_async_remote_copy(
                src_ref=x_ref.at[pl.ds(src_row, c)],
                dst_ref=out_ref.at[pl.ds(dst_row, c)],
                send_sem=x_send.at[k],
                recv_sem=x_recv.at[k],
                device_id=xn,
                device_id_type=pl.DeviceIdType.MESH,
            )
            r.start()
            x_rdmas.append(r)

        out_dmas = []
        y_rdmas = []
        for k in range(N_CHUNKS):
            slot = k % 2
            if k >= 2:
                out_dmas[k - 2].wait()
            d_in = pltpu.make_async_copy(
                x_ref.at[pl.ds(k * lc, lc)], stage.at[slot],
                in_sems.at[slot],
            )
            d_in.start()
            d_in.wait()
            d_out = pltpu.make_async_copy(
                stage.at[slot],
                out_ref.at[pl.ds(my_x * m_per + k * lc, lc)],
                out_sems.at[slot],
            )
            d_out.start()
            out_dmas.append(d_out)

            x_rdmas[k].wait_recv()
            row = (1 - my_x) * m_per + my_y * half + k * c
            r = pltpu.make_async_remote_copy(
                src_ref=out_ref.at[pl.ds(row, c)],
                dst_ref=out_ref.at[pl.ds(row, c)],
                send_sem=y_send.at[k],
                recv_sem=y_recv.at[k],
                device_id=yn,
                device_id_type=pl.DeviceIdType.MESH,
            )
            r.start()
            y_rdmas.append(r)

        for k in range(N_CHUNKS):
            x_rdmas[k].wait_send()
            y_rdmas[k].wait_send()
            y_rdmas[k].wait_recv()
        out_dmas[-2].wait()
        out_dmas[-1].wait()

    return pl.pallas_call(
        body,
        out_shape=jax.ShapeDtypeStruct((m_global, n), x.dtype),
        in_specs=[pl.BlockSpec(memory_space=pl.ANY)],
        out_specs=pl.BlockSpec(memory_space=pl.ANY),
        scratch_shapes=[
            pltpu.VMEM((2, lc, n), x.dtype),
            pltpu.SemaphoreType.DMA((2,)),
            pltpu.SemaphoreType.DMA((2,)),
            pltpu.SemaphoreType.DMA((N_CHUNKS,)),
            pltpu.SemaphoreType.DMA((N_CHUNKS,)),
            pltpu.SemaphoreType.DMA((N_CHUNKS,)),
            pltpu.SemaphoreType.DMA((N_CHUNKS,)),
        ],
        compiler_params=pltpu.CompilerParams(collective_id=0),
    )(x)


# device time: 920305 ns/iter; 4.6180x vs baseline; 1.0240x over previous
import jax
import jax.numpy as jnp
from jax import lax
from jax.experimental import pallas as pl
from jax.experimental.pallas import tpu as pltpu

N_CHUNKS = 32


def kernel(x):
    m_per, n = x.shape
    m_global = 2 * m_per
    half = m_per // 2
    c = half // N_CHUNKS
    lc = m_per // N_CHUNKS

    def body(x_ref, out_ref, stage, in_sems, out_sems,
             x_send, x_recv, y_send, y_recv):
        my_x = lax.axis_index("x")
        my_y = lax.axis_index("y")
        xn = (1 - my_x, my_y)
        yn = (my_x, 1 - my_y)

        barrier_sem = pltpu.get_barrier_semaphore()
        for nbr in (xn, yn):
            pl.semaphore_signal(
                barrier_sem, inc=1, device_id=nbr,
                device_id_type=pl.DeviceIdType.MESH,
            )
        pl.semaphore_wait(barrier_sem, 2)

        x_rdmas = []
        for k in range(N_CHUNKS):
            src_row = my_y * half + k * c
            dst_row = my_x * m_per + my_y * half + k * c
            r = pltpu.make_async_remote_copy(
                src_ref=x_ref.at[pl.ds(src_row, c)],
                dst_ref=out_ref.at[pl.ds(dst_row, c)],
                send_sem=x_send.at[k],
                recv_sem=x_recv.at[k],
                device_id=xn,
                device_id_type=pl.DeviceIdType.MESH,
            )
            r.start()
            x_rdmas.append(r)

        out_dmas = []
        y_rdmas = []
        for k in range(N_CHUNKS):
            slot = k % 2
            if k >= 2:
                out_dmas[k - 2].wait()
            d_in = pltpu.make_async_copy(
                x_ref.at[pl.ds(k * lc, lc)], stage.at[slot],
                in_sems.at[slot],
            )
            d_in.start()
            d_in.wait()
            d_out = pltpu.make_async_copy(
                stage.at[slot],
                out_ref.at[pl.ds(my_x * m_per + k * lc, lc)],
                out_sems.at[slot],
            )
            d_out.start()
            out_dmas.append(d_out)

            x_rdmas[k].wait_recv()
            row = (1 - my_x) * m_per + my_y * half + k * c
            r = pltpu.make_async_remote_copy(
                src_ref=out_ref.at[pl.ds(row, c)],
                dst_ref=out_ref.at[pl.ds(row, c)],
                send_sem=y_send.at[k],
                recv_sem=y_recv.at[k],
                device_id=yn,
                device_id_type=pl.DeviceIdType.MESH,
            )
            r.start()
            y_rdmas.append(r)

        for k in range(N_CHUNKS):
            x_rdmas[k].wait_send()
            y_rdmas[k].wait_send()
            y_rdmas[k].wait_recv()
        out_dmas[-2].wait()
        out_dmas[-1].wait()

    return pl.pallas_call(
        body,
        out_shape=jax.ShapeDtypeStruct((m_global, n), x.dtype),
        in_specs=[pl.BlockSpec(memory_space=pl.ANY)],
        out_specs=pl.BlockSpec(memory_space=pl.ANY),
        scratch_shapes=[
            pltpu.VMEM((2, lc, n), x.dtype),
            pltpu.SemaphoreType.DMA((2,)),
            pltpu.SemaphoreType.DMA((2,)),
            pltpu.SemaphoreType.DMA((N_CHUNKS,)),
            pltpu.SemaphoreType.DMA((N_CHUNKS,)),
            pltpu.SemaphoreType.DMA((N_CHUNKS,)),
            pltpu.SemaphoreType.DMA((N_CHUNKS,)),
        ],
        compiler_params=pltpu.CompilerParams(collective_id=0),
    )(x)
